# baseline (speedup 1.0000x reference)
import jax
import jax.numpy as jnp
from jax import lax
from jax.experimental import pallas as pl
from jax.experimental.pallas import tpu as pltpu
from jax.experimental.pallas import tpu_sc as plsc

R = 65536
S = 64
NB = S + 1
OUT = 2 * S + 1
L = 16
NW = 32
SLAB = 128
CH = 4
QUADS = SLAB // (CH * L)
SLABS = R // NW // SLAB
UPAD = 256
HIST_PAD = 0.01


def _body(bins_hbm, w_hbm, near_hbm, far_hbm, u_hbm,
          out_hbm,
          bins_v, w_v, near_v, far_v, u_v, cdf_v, newb_v, out_v):
    wid = lax.axis_index("s") * 2 + lax.axis_index("c")
    iota = lax.iota(jnp.int32, L)
    zf = jnp.zeros((L,), jnp.float32)
    zi = jnp.zeros((L,), jnp.int32)
    crow = [iota + c * L for c in range(CH)]
    pltpu.sync_copy(u_hbm, u_v)

    def slab(sl, _):
        base = wid * (SLABS * SLAB) + sl * SLAB
        pltpu.sync_copy(bins_hbm.at[pl.ds(base, SLAB)], bins_v)
        pltpu.sync_copy(w_hbm.at[pl.ds(base, SLAB)], w_v)
        pltpu.sync_copy(near_hbm.at[pl.ds(base, SLAB)], near_v)
        pltpu.sync_copy(far_hbm.at[pl.ds(base, SLAB)], far_v)

        def quad(q, _):
            row = [iota + (q * CH + c) * L for c in range(CH)]

            def cum(k, cs):
                fk = jnp.full((L,), k, jnp.int32)
                nxt = []
                for c in range(CH):
                    v = plsc.load_gather(w_v, [row[c], fk])
                    acc = cs[c] + (v + HIST_PAD)
                    plsc.store_scatter(cdf_v, [crow[c], fk + 1], acc)
                    nxt.append(acc)
                return tuple(nxt)
            wsum = lax.fori_loop(0, S, cum, (zf,) * CH, unroll=8)
            f0 = jnp.full((L,), 0, jnp.int32)
            fNB = jnp.full((L,), NB, jnp.int32)
            for c in range(CH):
                plsc.store_scatter(cdf_v, [crow[c], f0], zf)
                plsc.store_scatter(cdf_v, [crow[c], fNB], wsum[c] + 1.0)

            def step(_, st):
                nxt = []
                for c in range(CH):
                    kv, jv, c0 = st[c]
                    uw = plsc.load_gather(u_v, [jv]) * wsum[c]
                    c1 = plsc.load_gather(cdf_v, [crow[c], kv + 1])
                    adv = c1 <= uw
                    b0 = plsc.load_gather(bins_v, [row[c], kv])
                    b1 = plsc.load_gather(bins_v,
                                          [row[c], jnp.minimum(kv + 1, S)])
                    t = jnp.clip((uw - c0) / (c1 - c0), 0.0, 1.0)
                    plsc.store_scatter(newb_v,
                                       [crow[c], jnp.minimum(jv, NB - 1)],
                                       b0 + t * (b1 - b0),
                                       mask=(~adv) & (jv < NB))
                    nxt.append((kv + adv.astype(jnp.int32),
                                jv + (~adv).astype(jnp.int32),
                                jnp.where(adv, c1, c0)))
                return tuple(nxt)
            lax.fori_loop(0, S + NB + 1, step, ((zi, zi, zf),) * CH, unroll=4)

            near = [plsc.load_gather(near_v, [row[c]]) for c in range(CH)]
            span = [plsc.load_gather(far_v, [row[c]]) - near[c]
                    for c in range(CH)]
            big = jnp.float32(jnp.inf)

            def merge(pp, st):
                fp = jnp.full((L,), pp, jnp.int32)
                nxt = []
                for c in range(CH):
                    iv, jv = st[c]
                    a = plsc.load_gather(bins_v, [row[c], iv])
                    b = plsc.load_gather(newb_v, [crow[c], jv])
                    b = jnp.where(jv >= S, big, b)
                    take = a < b
                    plsc.store_scatter(out_v, [row[c], fp],
                                       near[c] + jnp.minimum(a, b) * span[c])
                    nxt.append((iv + take.astype(jnp.int32),
                                jv + (~take).astype(jnp.int32)))
                return tuple(nxt)
            lax.fori_loop(0, 2 * S, merge, ((zi, zi),) * CH, unroll=4)

            fS = jnp.full((L,), S, jnp.int32)
            fO = jnp.full((L,), 2 * S, jnp.int32)
            for c in range(CH):
                ends = jnp.maximum(plsc.load_gather(bins_v, [row[c], fS]),
                                   plsc.load_gather(newb_v, [crow[c], fS]))
                plsc.store_scatter(out_v, [row[c], fO],
                                   near[c] + ends * span[c])
            return 0

        lax.fori_loop(0, QUADS, quad, 0)
        pltpu.sync_copy(out_v, out_hbm.at[pl.ds(base, SLAB)])
        return 0

    lax.fori_loop(0, SLABS, slab, 0)


@jax.jit
def kernel(spacing_bins, weights, nears, fars):
    u = (jnp.linspace(0.0, 1.0 - 1.0 / NB, NB, dtype=jnp.float32)
         + 1.0 / (2 * NB))
    u_pad = jnp.zeros((UPAD,), jnp.float32).at[:NB].set(u)
    mesh = plsc.VectorSubcoreMesh(core_axis_name="c", subcore_axis_name="s")
    fn = pl.kernel(
        _body,
        out_type=jax.ShapeDtypeStruct((R, OUT), jnp.float32),
        mesh=mesh,
        compiler_params=pltpu.CompilerParams(needs_layout_passes=False),
        scratch_types=[
            pltpu.VMEM((SLAB, NB), jnp.float32),
            pltpu.VMEM((SLAB, S), jnp.float32),
            pltpu.VMEM((SLAB,), jnp.float32),
            pltpu.VMEM((SLAB,), jnp.float32),
            pltpu.VMEM((UPAD,), jnp.float32),
            pltpu.VMEM((CH * L, NB + 1), jnp.float32),
            pltpu.VMEM((CH * L, NB), jnp.float32),
            pltpu.VMEM((SLAB, OUT), jnp.float32),
        ],
    )
    return fn(spacing_bins, weights, nears.reshape(R), fars.reshape(R), u_pad)


# speedup vs baseline: 1.5404x; 1.5404x over previous
import jax
import jax.numpy as jnp
from jax import lax
from jax.experimental import pallas as pl
from jax.experimental.pallas import tpu as pltpu
from jax.experimental.pallas import tpu_sc as plsc

R = 65536
S = 64
NB = S + 1
OUT = 2 * S + 1
L = 16
NW = 32
SLAB = 128
CH = 2
NCOL = 132
QUADS = SLAB // (CH * L)
SLABS = R // NW // SLAB
HIST_PAD = 0.01
U0 = 1.0 / (2 * NB)
DU = (1.0 - 1.0 / NB) / (NB - 1)


def _body(bins_hbm, w_hbm, near_hbm, far_hbm,
          out_hbm,
          bins_v, w_v, near_v, far_v, cdf_v, newb_v, out_v):
    wid = lax.axis_index("s") * 2 + lax.axis_index("c")
    iota = lax.iota(jnp.int32, L)
    zf = jnp.zeros((L,), jnp.float32)
    zi = jnp.zeros((L,), jnp.int32)
    crow = [iota + c * L for c in range(CH)]

    def slab(sl, _):
        base = wid * (SLABS * SLAB) + sl * SLAB
        pltpu.sync_copy(bins_hbm.at[pl.ds(base, SLAB)], bins_v)
        pltpu.sync_copy(w_hbm.at[pl.ds(base, SLAB)], w_v)
        pltpu.sync_copy(near_hbm.at[pl.ds(base, SLAB)], near_v)
        pltpu.sync_copy(far_hbm.at[pl.ds(base, SLAB)], far_v)

        def quad(q, _):
            row = [iota + (q * CH + c) * L for c in range(CH)]

            def cum(k, cs):
                fk = jnp.full((L,), k, jnp.int32)
                nxt = []
                for c in range(CH):
                    v = plsc.load_gather(w_v, [row[c], fk])
                    acc = cs[c] + (v + HIST_PAD)
                    plsc.store_scatter(cdf_v, [crow[c], fk + 1], acc)
                    nxt.append(acc)
                return tuple(nxt)
            wsum = plsc.parallel_loop(0, S, carry=(zf,) * CH, unroll=8)(cum)
            f0 = jnp.full((L,), 0, jnp.int32)
            fNB = jnp.full((L,), NB, jnp.int32)
            inf = jnp.full((L,), jnp.inf, jnp.float32)
            for c in range(CH):
                plsc.store_scatter(cdf_v, [crow[c], f0], zf)
                plsc.store_scatter(cdf_v, [crow[c], fNB], inf)
            u0w = [jnp.full((L,), U0, jnp.float32) * wsum[c] for c in range(CH)]
            duw = [jnp.full((L,), DU, jnp.float32) * wsum[c] for c in range(CH)]
            b0i = [plsc.load_gather(bins_v, [row[c], zi]) for c in range(CH)]

            def step(_, st):
                nxt = []
                for c in range(CH):
                    kv, jv, c0, b0, jf = st[c]
                    uw = jf * duw[c] + u0w[c]
                    c1 = plsc.load_gather(cdf_v, [crow[c], kv + 1])
                    adv = c1 <= uw
                    b1 = plsc.load_gather(bins_v,
                                          [row[c], jnp.minimum(kv + 1, S)])
                    t = jnp.clip((uw - c0) / (c1 - c0), 0.0, 1.0)
                    plsc.store_scatter(newb_v, [crow[c], jv],
                                       b0 + t * (b1 - b0), mask=~adv)
                    nxt.append((kv + adv.astype(jnp.int32),
                                jv + (~adv).astype(jnp.int32),
                                jnp.where(adv, c1, c0),
                                jnp.where(adv, b1, b0),
                                jf + jnp.where(adv, 0.0, 1.0)))
                return tuple(nxt)
            plsc.parallel_loop(0, S + NB + 1,
                               carry=tuple((zi, zi, zf, b0i[c], zf)
                                           for c in range(CH)))(step)

            near = [plsc.load_gather(near_v, [row[c]]) for c in range(CH)]
            span = [plsc.load_gather(far_v, [row[c]]) - near[c]
                    for c in range(CH)]
            fS = jnp.full((L,), S, jnp.int32)
            fO = jnp.full((L,), 2 * S, jnp.int32)
            inf = jnp.full((L,), jnp.inf, jnp.float32)
            for c in range(CH):
                ends = jnp.maximum(plsc.load_gather(bins_v, [row[c], fS]),
                                   plsc.load_gather(newb_v, [crow[c], fS]))
                plsc.store_scatter(out_v, [row[c], fO],
                                   near[c] + ends * span[c])
                plsc.store_scatter(newb_v, [crow[c], fS], inf)

            def merge(pp, st):
                fp = jnp.full((L,), pp, jnp.int32)
                nxt = []
                for c in range(CH):
                    iv, jv = st[c]
                    a = plsc.load_gather(bins_v, [row[c], iv])
                    b = plsc.load_gather(newb_v, [crow[c], jv])
                    take = a < b
                    plsc.store_scatter(out_v, [row[c], fp],
                                       near[c] + jnp.minimum(a, b) * span[c])
                    nxt.append((iv + take.astype(jnp.int32),
                                jv + (~take).astype(jnp.int32)))
                return tuple(nxt)
            plsc.parallel_loop(0, 2 * S, carry=((zi, zi),) * CH,
                               unroll=4)(merge)
            return 0

        lax.fori_loop(0, QUADS, quad, 0)
        pltpu.sync_copy(out_v, out_hbm.at[pl.ds(base, SLAB)])
        return 0

    lax.fori_loop(0, SLABS, slab, 0)


@jax.jit
def kernel(spacing_bins, weights, nears, fars):
    mesh = plsc.VectorSubcoreMesh(core_axis_name="c", subcore_axis_name="s")
    fn = pl.kernel(
        _body,
        out_type=jax.ShapeDtypeStruct((R, OUT), jnp.float32),
        mesh=mesh,
        compiler_params=pltpu.CompilerParams(needs_layout_passes=False),
        scratch_types=[
            pltpu.VMEM((SLAB, NB), jnp.float32),
            pltpu.VMEM((SLAB, S), jnp.float32),
            pltpu.VMEM((SLAB,), jnp.float32),
            pltpu.VMEM((SLAB,), jnp.float32),
            pltpu.VMEM((CH * L, NB + 1), jnp.float32),
            pltpu.VMEM((CH * L, NCOL), jnp.float32),
            pltpu.VMEM((SLAB, OUT), jnp.float32),
        ],
    )
    return fn(spacing_bins, weights, nears.reshape(R), fars.reshape(R))


# speedup vs baseline: 1.6729x; 1.0861x over previous
import jax
import jax.numpy as jnp
from jax import lax
from jax.experimental import pallas as pl
from jax.experimental.pallas import tpu as pltpu
from jax.experimental.pallas import tpu_sc as plsc

R = 65536
S = 64
NB = S + 1
OUT = 2 * S + 1
L = 16
NW = 32
SLAB = 128
CH = 2
NCOL = 132
QUADS = SLAB // (CH * L)
SLABS = R // NW // SLAB
HIST_PAD = 0.01
U0 = 1.0 / (2 * NB)
DU = (1.0 - 1.0 / NB) / (NB - 1)


def _body(bins_hbm, w_hbm, near_hbm, far_hbm,
          out_hbm,
          bins_a, w_a, near_a, far_a, bins_b, w_b, near_b, far_b,
          cdf_v, newb_v, out_v, sem_a, sem_b):
    wid = lax.axis_index("s") * 2 + lax.axis_index("c")
    iota = lax.iota(jnp.int32, L)
    zf = jnp.zeros((L,), jnp.float32)
    zi = jnp.zeros((L,), jnp.int32)
    crow = [iota + c * L for c in range(CH)]
    bufs = ((bins_a, w_a, near_a, far_a, sem_a),
            (bins_b, w_b, near_b, far_b, sem_b))

    def in_copies(sl, buf):
        bins_v, w_v, near_v, far_v, sem = buf
        base = wid * (SLABS * SLAB) + sl * SLAB
        return (pltpu.make_async_copy(bins_hbm.at[pl.ds(base, SLAB)], bins_v, sem),
                pltpu.make_async_copy(w_hbm.at[pl.ds(base, SLAB)], w_v, sem),
                pltpu.make_async_copy(near_hbm.at[pl.ds(base, SLAB)], near_v, sem),
                pltpu.make_async_copy(far_hbm.at[pl.ds(base, SLAB)], far_v, sem))

    def start_in(sl, buf):
        for cp in in_copies(sl, buf):
            cp.start()

    def wait_in(sl, buf):
        for cp in in_copies(sl, buf):
            cp.wait()

    def compute(sl, buf):
        bins_v, w_v, near_v, far_v, _ = buf
        base = wid * (SLABS * SLAB) + sl * SLAB

        def quad(q, _):
            row = [iota + (q * CH + c) * L for c in range(CH)]

            def cum(k, cs):
                fk = jnp.full((L,), k, jnp.int32)
                nxt = []
                for c in range(CH):
                    v = plsc.load_gather(w_v, [row[c], fk])
                    acc = cs[c] + (v + HIST_PAD)
                    plsc.store_scatter(cdf_v, [crow[c], fk + 1], acc)
                    nxt.append(acc)
                return tuple(nxt)
            wsum = plsc.parallel_loop(0, S, carry=(zf,) * CH, unroll=8)(cum)
            fNB = jnp.full((L,), NB, jnp.int32)
            inf = jnp.full((L,), jnp.inf, jnp.float32)
            for c in range(CH):
                plsc.store_scatter(cdf_v, [crow[c], fNB], inf)
            u0w = [jnp.full((L,), U0, jnp.float32) * wsum[c] for c in range(CH)]
            duw = [jnp.full((L,), DU, jnp.float32) * wsum[c] for c in range(CH)]
            b0i = [plsc.load_gather(bins_v, [row[c], zi]) for c in range(CH)]

            def step(_, st):
                nxt = []
                for c in range(CH):
                    kv, jv, c0, b0, jf = st[c]
                    uw = jf * duw[c] + u0w[c]
                    c1 = plsc.load_gather(cdf_v, [crow[c], kv + 1])
                    adv = c1 <= uw
                    b1 = plsc.load_gather(bins_v,
                                          [row[c], jnp.minimum(kv + 1, S)])
                    t = jnp.clip((uw - c0) / (c1 - c0), 0.0, 1.0)
                    plsc.store_scatter(newb_v, [crow[c], jv],
                                       b0 + t * (b1 - b0), mask=~adv)
                    nxt.append((kv + adv.astype(jnp.int32),
                                jv + (~adv).astype(jnp.int32),
                                jnp.where(adv, c1, c0),
                                jnp.where(adv, b1, b0),
                                jf + jnp.where(adv, 0.0, 1.0)))
                return tuple(nxt)
            plsc.parallel_loop(0, S + NB + 1,
                               carry=tuple((zi, zi, zf, b0i[c], zf)
                                           for c in range(CH)))(step)

            near = [plsc.load_gather(near_v, [row[c]]) for c in range(CH)]
            span = [plsc.load_gather(far_v, [row[c]]) - near[c]
                    for c in range(CH)]
            fS = jnp.full((L,), S, jnp.int32)
            fO = jnp.full((L,), 2 * S, jnp.int32)
            for c in range(CH):
                ends = jnp.maximum(plsc.load_gather(bins_v, [row[c], fS]),
                                   plsc.load_gather(newb_v, [crow[c], fS]))
                plsc.store_scatter(out_v, [row[c], fO],
                                   near[c] + ends * span[c])
                plsc.store_scatter(newb_v, [crow[c], fS], inf)

            def merge(pp, st):
                fp = jnp.full((L,), pp, jnp.int32)
                nxt = []
                for c in range(CH):
                    iv, jv = st[c]
                    a = plsc.load_gather(bins_v, [row[c], iv])
                    b = plsc.load_gather(newb_v, [crow[c], jv])
                    take = a < b
                    plsc.store_scatter(out_v, [row[c], fp],
                                       near[c] + jnp.minimum(a, b) * span[c])
                    nxt.append((iv + take.astype(jnp.int32),
                                jv + (~take).astype(jnp.int32)))
                return tuple(nxt)
            plsc.parallel_loop(0, 2 * S, carry=((zi, zi),) * CH,
                               unroll=4)(merge)
            return 0

        lax.fori_loop(0, QUADS, quad, 0)
        pltpu.sync_copy(out_v, out_hbm.at[pl.ds(base, SLAB)])

    start_in(0, bufs[0])

    def pair(sp, _):
        s0 = 2 * sp
        wait_in(s0, bufs[0])
        start_in(s0 + 1, bufs[1])
        compute(s0, bufs[0])
        wait_in(s0 + 1, bufs[1])

        @pl.when(sp < SLABS // 2 - 1)
        def _():
            start_in(s0 + 2, bufs[0])

        compute(s0 + 1, bufs[1])
        return 0

    lax.fori_loop(0, SLABS // 2, pair, 0)


@jax.jit
def kernel(spacing_bins, weights, nears, fars):
    mesh = plsc.VectorSubcoreMesh(core_axis_name="c", subcore_axis_name="s")
    in_set = [
        pltpu.VMEM((SLAB, NB), jnp.float32),
        pltpu.VMEM((SLAB, S), jnp.float32),
        pltpu.VMEM((SLAB,), jnp.float32),
        pltpu.VMEM((SLAB,), jnp.float32),
    ]
    fn = pl.kernel(
        _body,
        out_type=jax.ShapeDtypeStruct((R, OUT), jnp.float32),
        mesh=mesh,
        compiler_params=pltpu.CompilerParams(needs_layout_passes=False),
        scratch_types=in_set + in_set + [
            pltpu.VMEM((CH * L, NB + 1), jnp.float32),
            pltpu.VMEM((CH * L, NCOL), jnp.float32),
            pltpu.VMEM((SLAB, OUT), jnp.float32),
            pltpu.SemaphoreType.DMA,
            pltpu.SemaphoreType.DMA,
        ],
    )
    return fn(spacing_bins, weights, nears.reshape(R), fars.reshape(R))
